# Initial kernel scaffold; baseline (speedup 1.0000x reference)
import functools

import jax
import jax.numpy as jnp
from jax import lax
from jax.experimental import pallas as pl
from jax.experimental.pallas import tpu as pltpu
from jax.experimental.pallas import tpu_sc as plsc

NC = 2
NS = 16
NW = NC * NS


def _mesh():
    return plsc.VectorSubcoreMesh(
        core_axis_name="c", subcore_axis_name="s", num_cores=NC, num_subcores=NS
    )


_Z16 = functools.partial(jnp.zeros, (16,), jnp.float32)


def _hist(dst_r, n_pad):
    nw, nchunk, c = dst_r.shape
    per_tile = n_pad // NS

    @functools.partial(
        pl.kernel,
        out_type=jax.ShapeDtypeStruct((NC * n_pad,), jnp.float32),
        mesh=_mesh(),
        scratch_types=[
            pltpu.VMEM((nchunk, c), jnp.int32),
            pltpu.VMEM((c,), jnp.float32),
            pltpu.VMEM((per_tile,), jnp.float32),
            pltpu.VMEM_SHARED((n_pad,), jnp.float32),
        ],
    )
    def hist(dst_hbm, out_hbm, dst_v, ones_v, zbuf_v, deg_sh):
        cid = lax.axis_index("c")
        sid = lax.axis_index("s")
        wid = cid * NS + sid
        pltpu.sync_copy(dst_hbm.at[wid], dst_v)
        for k in range(c // 16):
            ones_v[pl.ds(16 * k, 16)] = jnp.ones((16,), jnp.float32)
        for k in range(per_tile // 16):
            zbuf_v[pl.ds(16 * k, 16)] = _Z16()
        pltpu.sync_copy(zbuf_v, deg_sh.at[pl.ds(sid * per_tile, per_tile)])
        plsc.subcore_barrier()

        def body(j, carry):
            pltpu.sync_copy(ones_v, deg_sh.at[dst_v.at[j]], add=True)
            return carry

        lax.fori_loop(0, nchunk, body, 0)
        plsc.subcore_barrier()
        pltpu.sync_copy(
            deg_sh.at[pl.ds(sid * per_tile, per_tile)],
            out_hbm.at[pl.ds(cid * n_pad + sid * per_tile, per_tile)],
        )

    return hist(dst_r)


def _spmm(g, src_r, dst_r):
    n, f = g.shape
    nw, nchunk, c = src_r.shape
    rows_per_tile = n // NS
    zrows = 125

    @functools.partial(
        pl.kernel,
        out_type=jax.ShapeDtypeStruct((NC * n, f), jnp.float32),
        mesh=_mesh(),
        scratch_types=[
            pltpu.VMEM((nchunk, c), jnp.int32),
            pltpu.VMEM((nchunk, c), jnp.int32),
            pltpu.VMEM((c, f), jnp.float32),
            pltpu.VMEM((zrows, f), jnp.float32),
            pltpu.VMEM_SHARED((n, f), jnp.float32),
            pltpu.SemaphoreType.DMA,
        ],
    )
    def spmm(g_hbm, src_hbm, dst_hbm, out_hbm, src_v, dst_v, rows_v, zbuf_v, acc_sh, sem):
        cid = lax.axis_index("c")
        sid = lax.axis_index("s")
        wid = cid * NS + sid
        pltpu.sync_copy(src_hbm.at[wid], src_v)
        pltpu.sync_copy(dst_hbm.at[wid], dst_v)

        def zfill(i, carry):
            for k in range(f // 16):
                zbuf_v[i, pl.ds(16 * k, 16)] = _Z16()
            return carry

        lax.fori_loop(0, zrows, zfill, 0)

        def zcopy(k, carry):
            pltpu.sync_copy(
                zbuf_v, acc_sh.at[pl.ds(sid * rows_per_tile + k * zrows, zrows)]
            )
            return carry

        lax.fori_loop(0, rows_per_tile // zrows, zcopy, 0)
        plsc.subcore_barrier()

        def body(j, carry):
            pltpu.async_copy(g_hbm.at[src_v.at[j]], rows_v, sem).wait()
            pltpu.sync_copy(rows_v, acc_sh.at[dst_v.at[j]], add=True)
            return carry

        lax.fori_loop(0, nchunk, body, 0)
        plsc.subcore_barrier()

        def wout(k, carry):
            r0 = sid * rows_per_tile + k * zrows
            pltpu.sync_copy(
                acc_sh.at[pl.ds(r0, zrows)], out_hbm.at[pl.ds(cid * n + r0, zrows)]
            )
            return carry

        lax.fori_loop(0, rows_per_tile // zrows, wout, 0)

    return spmm(g, src_r, dst_r)


def _decode(z, a_r, b_r):
    n, f = z.shape
    nw, ncd, cd = a_r.shape

    @functools.partial(
        pl.kernel,
        out_type=jax.ShapeDtypeStruct((NW * ncd * cd,), jnp.float32),
        mesh=_mesh(),
        scratch_types=[
            pltpu.VMEM((ncd, cd), jnp.int32),
            pltpu.VMEM((ncd, cd), jnp.int32),
            pltpu.VMEM((cd, f), jnp.float32),
            pltpu.VMEM((cd, f), jnp.float32),
            pltpu.VMEM((cd,), jnp.float32),
            pltpu.SemaphoreType.DMA,
        ],
    )
    def decode(z_hbm, a_hbm, b_hbm, out_hbm, a_v, b_v, za_v, zb_v, lg_v, sem):
        cid = lax.axis_index("c")
        sid = lax.axis_index("s")
        wid = cid * NS + sid
        pltpu.sync_copy(a_hbm.at[wid], a_v)
        pltpu.sync_copy(b_hbm.at[wid], b_v)
        iota16 = lax.iota(jnp.int32, 16)

        def chunk(j, carry):
            pltpu.async_copy(z_hbm.at[a_v.at[j]], za_v, sem).wait()
            pltpu.async_copy(z_hbm.at[b_v.at[j]], zb_v, sem).wait()
            for grp in range(cd // 16):
                rows = iota16 + (16 * grp)

                def col(jj, acc):
                    cols = jnp.full((16,), jj, jnp.int32)
                    return acc + plsc.load_gather(za_v, [rows, cols]) * plsc.load_gather(
                        zb_v, [rows, cols]
                    )

                lg_v[pl.ds(16 * grp, 16)] = lax.fori_loop(0, f, col, _Z16())
            pltpu.sync_copy(lg_v, out_hbm.at[pl.ds(wid * ncd * cd + j * cd, cd)])
            return carry

        lax.fori_loop(0, ncd, chunk, 0)

    return decode(z, a_r, b_r)


def _tc1(x, w1, d0, d1):
    n, k = x.shape
    h = w1.shape[1]
    r = 2000

    def body(x_ref, w_ref, d0_ref, d1_ref, g_ref, dinv_ref):
        deg = 1.0 + d0_ref[...] + d1_ref[...]
        dinv = lax.rsqrt(deg)
        hm = jnp.dot(x_ref[...], w_ref[...], preferred_element_type=jnp.float32)
        g_ref[...] = dinv * hm
        dinv_ref[...] = dinv

    return pl.pallas_call(
        body,
        grid=(n // r,),
        in_specs=[
            pl.BlockSpec((r, k), lambda i: (i, 0)),
            pl.BlockSpec((k, h), lambda i: (0, 0)),
            pl.BlockSpec((r, 1), lambda i: (i, 0)),
            pl.BlockSpec((r, 1), lambda i: (i, 0)),
        ],
        out_specs=[
            pl.BlockSpec((r, h), lambda i: (i, 0)),
            pl.BlockSpec((r, 1), lambda i: (i, 0)),
        ],
        out_shape=[
            jax.ShapeDtypeStruct((n, h), jnp.float32),
            jax.ShapeDtypeStruct((n, 1), jnp.float32),
        ],
    )(x, w1, d0, d1)


def _tc2(g1, s0, s1, dinv, b1, w2):
    n, h = g1.shape
    dout = w2.shape[1]
    r = 2000

    def body(g_ref, s0_ref, s1_ref, di_ref, b_ref, w_ref, o_ref):
        di = di_ref[...]
        u = jnp.maximum(di * (s0_ref[...] + s1_ref[...] + g_ref[...]) + b_ref[...], 0.0)
        o_ref[...] = di * jnp.dot(u, w_ref[...], preferred_element_type=jnp.float32)

    return pl.pallas_call(
        body,
        grid=(n // r,),
        in_specs=[
            pl.BlockSpec((r, h), lambda i: (i, 0)),
            pl.BlockSpec((r, h), lambda i: (i, 0)),
            pl.BlockSpec((r, h), lambda i: (i, 0)),
            pl.BlockSpec((r, 1), lambda i: (i, 0)),
            pl.BlockSpec((1, h), lambda i: (0, 0)),
            pl.BlockSpec((h, dout), lambda i: (0, 0)),
        ],
        out_specs=pl.BlockSpec((r, dout), lambda i: (i, 0)),
        out_shape=jax.ShapeDtypeStruct((n, dout), jnp.float32),
    )(g1, s0, s1, dinv, b1, w2)


def _tc3(g2, s0, s1, dinv, b2):
    n, dout = g2.shape
    r = 2000

    def body(g_ref, s0_ref, s1_ref, di_ref, b_ref, o_ref):
        o_ref[...] = (
            di_ref[...] * (s0_ref[...] + s1_ref[...] + g_ref[...]) + b_ref[...]
        )

    return pl.pallas_call(
        body,
        grid=(n // r,),
        in_specs=[
            pl.BlockSpec((r, dout), lambda i: (i, 0)),
            pl.BlockSpec((r, dout), lambda i: (i, 0)),
            pl.BlockSpec((r, dout), lambda i: (i, 0)),
            pl.BlockSpec((r, 1), lambda i: (i, 0)),
            pl.BlockSpec((1, dout), lambda i: (0, 0)),
        ],
        out_specs=pl.BlockSpec((r, dout), lambda i: (i, 0)),
        out_shape=jax.ShapeDtypeStruct((n, dout), jnp.float32),
    )(g2, s0, s1, dinv, b2)


def kernel(x, edge_index, edge_label_index, W1, b1, W2, b2):
    n, _ = x.shape
    e = edge_index.shape[1]
    l = edge_label_index.shape[1]

    c = 80
    epw = e // NW
    nchunk = epw // c
    src_r = edge_index[0].reshape(NW, nchunk, c)
    dst_r = edge_index[1].reshape(NW, nchunk, c)

    n_pad = 640 * NS
    degp = _hist(dst_r, n_pad).reshape(NC, n_pad)
    d0 = degp[0, :n, None]
    d1 = degp[1, :n, None]

    g1, dinv = _tc1(x, W1, d0, d1)
    s1 = _spmm(g1, src_r, dst_r).reshape(NC, n, -1)
    g2 = _tc2(g1, s1[0], s1[1], dinv, b1.reshape(1, -1), W2)
    s2 = _spmm(g2, src_r, dst_r).reshape(NC, n, -1)
    z = _tc3(g2, s2[0], s2[1], dinv, b2.reshape(1, -1))

    cd = 128
    ppw = l // NW
    ncd = 5
    eli = edge_label_index.reshape(2, NW, ppw)
    eli = jnp.pad(eli, ((0, 0), (0, 0), (0, ncd * cd - ppw)))
    a_r = eli[0].reshape(NW, ncd, cd)
    b_r = eli[1].reshape(NW, ncd, cd)
    lp = _decode(z, a_r, b_r)
    return lp.reshape(NW, ncd * cd)[:, :ppw].reshape(l)


# speedup vs baseline: 13.4260x; 13.4260x over previous
import functools

import jax
import jax.numpy as jnp
from jax import lax
from jax.experimental import pallas as pl
from jax.experimental.pallas import tpu as pltpu
from jax.experimental.pallas import tpu_sc as plsc

NC = 2
NS = 16
NW = NC * NS


def _mesh():
    return plsc.VectorSubcoreMesh(
        core_axis_name="c", subcore_axis_name="s", num_cores=NC, num_subcores=NS
    )


_SC_PARAMS = pltpu.CompilerParams(
    use_tc_tiling_on_sc=False, needs_layout_passes=False
)


_Z16 = functools.partial(jnp.zeros, (16,), jnp.float32)


def _hist(dst_r, n_pad):
    nw, nchunk, c = dst_r.shape
    per_tile = n_pad // NS

    @functools.partial(
        pl.kernel,
        out_type=jax.ShapeDtypeStruct((NC * n_pad,), jnp.float32),
        mesh=_mesh(),
        compiler_params=_SC_PARAMS,
        scratch_types=[
            pltpu.VMEM((nchunk, c), jnp.int32),
            pltpu.VMEM((c,), jnp.float32),
            pltpu.VMEM((per_tile,), jnp.float32),
            pltpu.VMEM_SHARED((n_pad,), jnp.float32),
        ],
    )
    def hist(dst_hbm, out_hbm, dst_v, ones_v, zbuf_v, deg_sh):
        cid = lax.axis_index("c")
        sid = lax.axis_index("s")
        wid = cid * NS + sid
        pltpu.sync_copy(dst_hbm.at[wid], dst_v)
        for k in range(c // 16):
            ones_v[pl.ds(16 * k, 16)] = jnp.ones((16,), jnp.float32)
        for k in range(per_tile // 16):
            zbuf_v[pl.ds(16 * k, 16)] = _Z16()
        pltpu.sync_copy(zbuf_v, deg_sh.at[pl.ds(sid * per_tile, per_tile)])
        plsc.subcore_barrier()

        def body(j, carry):
            pltpu.sync_copy(ones_v, deg_sh.at[dst_v.at[j]], add=True)
            return carry

        lax.fori_loop(0, nchunk, body, 0)
        plsc.subcore_barrier()
        pltpu.sync_copy(
            deg_sh.at[pl.ds(sid * per_tile, per_tile)],
            out_hbm.at[pl.ds(cid * n_pad + sid * per_tile, per_tile)],
        )

    return hist(dst_r)


def _spmm(g2n, src_r, dst_r, n_pad):
    _, fh = g2n.shape
    nw, nchunk, c = src_r.shape
    rows_per_tile = n_pad // NS
    zrows = 128

    @functools.partial(
        pl.kernel,
        out_type=jax.ShapeDtypeStruct((NC * n_pad, fh), jnp.float32),
        mesh=_mesh(),
        compiler_params=_SC_PARAMS,
        scratch_types=[
            pltpu.VMEM((nchunk, c), jnp.int32),
            pltpu.VMEM((nchunk, c), jnp.int32),
            pltpu.VMEM((c, fh), jnp.float32),
            pltpu.VMEM((zrows, fh), jnp.float32),
            pltpu.VMEM_SHARED((n_pad, fh), jnp.float32),
            pltpu.SemaphoreType.DMA,
        ],
    )
    def spmm(g_hbm, src_hbm, dst_hbm, out_hbm, src_v, dst_v, rows_v, zbuf_v, acc_sh, sem):
        cid = lax.axis_index("c")
        sid = lax.axis_index("s")
        wid = cid * NS + sid
        pltpu.sync_copy(src_hbm.at[wid], src_v)
        pltpu.sync_copy(dst_hbm.at[wid], dst_v)

        def zfill(i, carry):
            for k in range(fh // 16):
                zbuf_v[i, pl.ds(16 * k, 16)] = _Z16()
            return carry

        lax.fori_loop(0, zrows, zfill, 0)

        def zcopy(k, carry):
            pltpu.sync_copy(
                zbuf_v, acc_sh.at[pl.ds(sid * rows_per_tile + k * zrows, zrows)]
            )
            return carry

        lax.fori_loop(0, rows_per_tile // zrows, zcopy, 0)
        plsc.subcore_barrier()

        def body(j, carry):
            pltpu.async_copy(g_hbm.at[src_v.at[j]], rows_v, sem).wait()
            pltpu.sync_copy(rows_v, acc_sh.at[dst_v.at[j]], add=True)
            return carry

        lax.fori_loop(0, nchunk, body, 0)
        plsc.subcore_barrier()

        def wout(k, carry):
            r0 = sid * rows_per_tile + k * zrows
            pltpu.sync_copy(
                acc_sh.at[pl.ds(r0, zrows)], out_hbm.at[pl.ds(cid * n_pad + r0, zrows)]
            )
            return carry

        lax.fori_loop(0, rows_per_tile // zrows, wout, 0)

    return spmm(g2n, src_r, dst_r)


def _spmm_full(g, src_r2, dst_r2, n, n_pad):
    f = g.shape[1]
    fh = f // 2
    g2n = jnp.concatenate([g[:, :fh], g[:, fh:]], axis=0)
    out = _spmm(g2n, src_r2, dst_r2, n_pad).reshape(NC, n_pad, fh)[:, :n]
    return jnp.concatenate([out[0], out[1]], axis=1)


def _decode(z, a_r, b_r):
    n, f = z.shape
    nw, ncd, cd = a_r.shape

    @functools.partial(
        pl.kernel,
        out_type=jax.ShapeDtypeStruct((NW * ncd * cd,), jnp.float32),
        mesh=_mesh(),
        compiler_params=_SC_PARAMS,
        scratch_types=[
            pltpu.VMEM((ncd, cd), jnp.int32),
            pltpu.VMEM((ncd, cd), jnp.int32),
            pltpu.VMEM((cd, f), jnp.float32),
            pltpu.VMEM((cd, f), jnp.float32),
            pltpu.VMEM((cd,), jnp.float32),
            pltpu.SemaphoreType.DMA,
        ],
    )
    def decode(z_hbm, a_hbm, b_hbm, out_hbm, a_v, b_v, za_v, zb_v, lg_v, sem):
        cid = lax.axis_index("c")
        sid = lax.axis_index("s")
        wid = cid * NS + sid
        pltpu.sync_copy(a_hbm.at[wid], a_v)
        pltpu.sync_copy(b_hbm.at[wid], b_v)
        iota16 = lax.iota(jnp.int32, 16)

        def chunk(j, carry):
            pltpu.async_copy(z_hbm.at[a_v.at[j]], za_v, sem).wait()
            pltpu.async_copy(z_hbm.at[b_v.at[j]], zb_v, sem).wait()
            for grp in range(cd // 16):
                rows = iota16 + (16 * grp)

                def col(jj, acc):
                    cols = jnp.full((16,), jj, jnp.int32)
                    return acc + plsc.load_gather(za_v, [rows, cols]) * plsc.load_gather(
                        zb_v, [rows, cols]
                    )

                lg_v[pl.ds(16 * grp, 16)] = lax.fori_loop(0, f, col, _Z16())
            pltpu.sync_copy(lg_v, out_hbm.at[pl.ds(wid * ncd * cd + j * cd, cd)])
            return carry

        lax.fori_loop(0, ncd, chunk, 0)

    return decode(z, a_r, b_r)


def _tc1(x, w1, d0, d1):
    n, k = x.shape
    h = w1.shape[1]
    r = 2000

    def body(x_ref, w_ref, d0_ref, d1_ref, g_ref, dinv_ref):
        deg = 1.0 + d0_ref[...] + d1_ref[...]
        dinv = lax.rsqrt(deg)
        hm = jnp.dot(x_ref[...], w_ref[...], preferred_element_type=jnp.float32)
        g_ref[...] = dinv * hm
        dinv_ref[...] = dinv

    return pl.pallas_call(
        body,
        grid=(n // r,),
        in_specs=[
            pl.BlockSpec((r, k), lambda i: (i, 0)),
            pl.BlockSpec((k, h), lambda i: (0, 0)),
            pl.BlockSpec((r, 1), lambda i: (i, 0)),
            pl.BlockSpec((r, 1), lambda i: (i, 0)),
        ],
        out_specs=[
            pl.BlockSpec((r, h), lambda i: (i, 0)),
            pl.BlockSpec((r, 1), lambda i: (i, 0)),
        ],
        out_shape=[
            jax.ShapeDtypeStruct((n, h), jnp.float32),
            jax.ShapeDtypeStruct((n, 1), jnp.float32),
        ],
    )(x, w1, d0, d1)


def _tc2(g1, s, dinv, b1, w2):
    n, h = g1.shape
    dout = w2.shape[1]
    r = 2000

    def body(g_ref, s_ref, di_ref, b_ref, w_ref, o_ref):
        di = di_ref[...]
        u = jnp.maximum(di * (s_ref[...] + g_ref[...]) + b_ref[...], 0.0)
        o_ref[...] = di * jnp.dot(u, w_ref[...], preferred_element_type=jnp.float32)

    return pl.pallas_call(
        body,
        grid=(n // r,),
        in_specs=[
            pl.BlockSpec((r, h), lambda i: (i, 0)),
            pl.BlockSpec((r, h), lambda i: (i, 0)),
            pl.BlockSpec((r, 1), lambda i: (i, 0)),
            pl.BlockSpec((1, h), lambda i: (0, 0)),
            pl.BlockSpec((h, dout), lambda i: (0, 0)),
        ],
        out_specs=pl.BlockSpec((r, dout), lambda i: (i, 0)),
        out_shape=jax.ShapeDtypeStruct((n, dout), jnp.float32),
    )(g1, s, dinv, b1, w2)


def _tc3(g2, s, dinv, b2):
    n, dout = g2.shape
    r = 2000

    def body(g_ref, s_ref, di_ref, b_ref, o_ref):
        o_ref[...] = di_ref[...] * (s_ref[...] + g_ref[...]) + b_ref[...]

    return pl.pallas_call(
        body,
        grid=(n // r,),
        in_specs=[
            pl.BlockSpec((r, dout), lambda i: (i, 0)),
            pl.BlockSpec((r, dout), lambda i: (i, 0)),
            pl.BlockSpec((r, 1), lambda i: (i, 0)),
            pl.BlockSpec((1, dout), lambda i: (0, 0)),
        ],
        out_specs=pl.BlockSpec((r, dout), lambda i: (i, 0)),
        out_shape=jax.ShapeDtypeStruct((n, dout), jnp.float32),
    )(g2, s, dinv, b2)


def kernel(x, edge_index, edge_label_index, W1, b1, W2, b2):
    n, _ = x.shape
    e = edge_index.shape[1]
    l = edge_label_index.shape[1]

    c = 80
    n_pad = 640 * NS

    ept_h = e // NW
    dst_r = edge_index[1].reshape(NW, ept_h // c, c)
    degp = _hist(dst_r, n_pad).reshape(NC, n_pad)
    d0 = degp[0, :n, None]
    d1 = degp[1, :n, None]

    ept = e // NS
    src16 = edge_index[0].reshape(NS, ept // c, c)
    src_r2 = jnp.concatenate([src16, src16 + n], axis=0)
    dst16 = edge_index[1].reshape(NS, ept // c, c)
    dst_r2 = jnp.concatenate([dst16, dst16], axis=0)

    g1, dinv = _tc1(x, W1, d0, d1)
    s1 = _spmm_full(g1, src_r2, dst_r2, n, n_pad)
    g2 = _tc2(g1, s1, dinv, b1.reshape(1, -1), W2)
    s2 = _spmm_full(g2, src_r2, dst_r2, n, n_pad)
    z = _tc3(g2, s2, dinv, b2.reshape(1, -1))

    cd = 128
    ppw = l // NW
    ncd = 5
    eli = edge_label_index.reshape(2, NW, ppw)
    eli = jnp.pad(eli, ((0, 0), (0, 0), (0, ncd * cd - ppw)))
    a_r = eli[0].reshape(NW, ncd, cd)
    b_r = eli[1].reshape(NW, ncd, cd)
    lp = _decode(z, a_r, b_r)
    return lp.reshape(NW, ncd * cd)[:, :ppw].reshape(l)


# speedup vs baseline: 23.2659x; 1.7329x over previous
import functools

import jax
import jax.numpy as jnp
from jax import lax
from jax.experimental import pallas as pl
from jax.experimental.pallas import tpu as pltpu
from jax.experimental.pallas import tpu_sc as plsc

NC = 2
NS = 16
NW = NC * NS


def _mesh():
    return plsc.VectorSubcoreMesh(
        core_axis_name="c", subcore_axis_name="s", num_cores=NC, num_subcores=NS
    )


_SC_PARAMS = pltpu.CompilerParams(
    use_tc_tiling_on_sc=False, needs_layout_passes=False
)


_Z16 = functools.partial(jnp.zeros, (16,), jnp.float32)


def _hist(dst_r, n_pad):
    nw, nchunk, c = dst_r.shape
    per_tile = n_pad // NS

    @functools.partial(
        pl.kernel,
        out_type=jax.ShapeDtypeStruct((NC * n_pad,), jnp.float32),
        mesh=_mesh(),
        compiler_params=_SC_PARAMS,
        scratch_types=[
            pltpu.VMEM((nchunk, c), jnp.int32),
            pltpu.VMEM((c,), jnp.float32),
            pltpu.VMEM((per_tile,), jnp.float32),
            pltpu.VMEM_SHARED((n_pad,), jnp.float32),
        ],
    )
    def hist(dst_hbm, out_hbm, dst_v, ones_v, zbuf_v, deg_sh):
        cid = lax.axis_index("c")
        sid = lax.axis_index("s")
        wid = cid * NS + sid
        pltpu.sync_copy(dst_hbm.at[wid], dst_v)
        for k in range(c // 16):
            ones_v[pl.ds(16 * k, 16)] = jnp.ones((16,), jnp.float32)
        for k in range(per_tile // 16):
            zbuf_v[pl.ds(16 * k, 16)] = _Z16()
        pltpu.sync_copy(zbuf_v, deg_sh.at[pl.ds(sid * per_tile, per_tile)])
        plsc.subcore_barrier()

        def body(j, carry):
            pltpu.sync_copy(ones_v, deg_sh.at[dst_v.at[j]], add=True)
            return carry

        lax.fori_loop(0, nchunk, body, 0)
        plsc.subcore_barrier()
        pltpu.sync_copy(
            deg_sh.at[pl.ds(sid * per_tile, per_tile)],
            out_hbm.at[pl.ds(cid * n_pad + sid * per_tile, per_tile)],
        )

    return hist(dst_r)


def _spmm(g2n, srcf, dstf, nchunk, c, n_pad):
    _, fh = g2n.shape
    rows_per_tile = n_pad // NS
    zrows = 32
    kk = 5
    nblk = nchunk // kk

    @functools.partial(
        pl.kernel,
        out_type=jax.ShapeDtypeStruct((NC * n_pad, fh), jnp.float32),
        mesh=_mesh(),
        compiler_params=_SC_PARAMS,
        scratch_types=[
            pltpu.VMEM((3 * kk, c), jnp.int32),
            pltpu.VMEM((3 * kk, c), jnp.int32),
            pltpu.VMEM((2 * kk * c, fh), jnp.float32),
            pltpu.VMEM((zrows, fh), jnp.float32),
            pltpu.VMEM_SHARED((n_pad, fh), jnp.float32),
            pltpu.SemaphoreType.DMA,
            pltpu.SemaphoreType.DMA,
            pltpu.SemaphoreType.DMA,
        ],
    )
    def spmm(g_hbm, src_hbm, dst_hbm, out_hbm, src_ib, dst_ib, rows_v, zbuf_v,
             acc_sh, gsem, ssem, isem):
        cid = lax.axis_index("c")
        sid = lax.axis_index("s")
        wid = cid * NS + sid
        base = wid * nchunk

        def zfill(i, carry):
            for k in range(fh // 16):
                zbuf_v[i, pl.ds(16 * k, 16)] = _Z16()
            return carry

        lax.fori_loop(0, zrows, zfill, 0)

        def zcopy(k, carry):
            pltpu.sync_copy(
                zbuf_v, acc_sh.at[pl.ds(sid * rows_per_tile + k * zrows, zrows)]
            )
            return carry

        lax.fori_loop(0, rows_per_tile // zrows, zcopy, 0)
        plsc.subcore_barrier()

        pltpu.sync_copy(src_hbm.at[pl.ds(base, kk)], src_ib.at[pl.ds(0, kk)])
        pltpu.sync_copy(dst_hbm.at[pl.ds(base, kk)], dst_ib.at[pl.ds(0, kk)])
        pltpu.async_copy(src_hbm.at[pl.ds(base + kk, kk)], src_ib.at[pl.ds(kk, kk)], isem)
        pltpu.async_copy(dst_hbm.at[pl.ds(base + kk, kk)], dst_ib.at[pl.ds(kk, kk)], isem)
        for b in range(kk):
            pltpu.async_copy(g_hbm.at[src_ib.at[b]], rows_v.at[pl.ds(b * c, c)], gsem)

        def block(t, carry):
            rg = lax.rem(t, 2) * kk
            g0 = lax.rem(t, 3) * kk
            g1 = lax.rem(t + 1, 3) * kk
            g2 = lax.rem(t + 2, 3) * kk
            for b in range(kk):
                pltpu.make_async_copy(
                    g_hbm.at[src_ib.at[0]], rows_v.at[pl.ds(0, c)], gsem
                ).wait()

            @pl.when(t >= 1)
            def _():
                for b in range(kk):
                    pltpu.make_async_copy(
                        g_hbm.at[src_ib.at[0]], rows_v.at[pl.ds(0, c)], ssem
                    ).wait()

            @pl.when(t + 2 < nblk)
            def _():
                pltpu.async_copy(
                    src_hbm.at[pl.ds(base + (t + 2) * kk, kk)],
                    src_ib.at[pl.ds(g2, kk)], isem,
                )
                pltpu.async_copy(
                    dst_hbm.at[pl.ds(base + (t + 2) * kk, kk)],
                    dst_ib.at[pl.ds(g2, kk)], isem,
                )

            for b in range(kk):
                pltpu.async_copy(
                    rows_v.at[pl.ds((rg + b) * c, c)],
                    acc_sh.at[dst_ib.at[g0 + b]], ssem, add=True,
                )

            @pl.when(t + 1 < nblk)
            def _():
                for b in range(2):
                    pltpu.make_async_copy(
                        src_hbm.at[pl.ds(0, kk)], src_ib.at[pl.ds(0, kk)], isem
                    ).wait()
                for b in range(kk):
                    pltpu.async_copy(
                        g_hbm.at[src_ib.at[g1 + b]],
                        rows_v.at[pl.ds((kk - rg + b) * c, c)], gsem,
                    )

            return carry

        lax.fori_loop(0, nblk, block, 0)
        for b in range(kk):
            pltpu.make_async_copy(
                g_hbm.at[src_ib.at[0]], rows_v.at[pl.ds(0, c)], ssem
            ).wait()
        plsc.subcore_barrier()

        def wout(k, carry):
            r0 = sid * rows_per_tile + k * 128
            pltpu.sync_copy(
                acc_sh.at[pl.ds(r0, 128)], out_hbm.at[pl.ds(cid * n_pad + r0, 128)]
            )
            return carry

        lax.fori_loop(0, rows_per_tile // 128, wout, 0)

    return spmm(g2n, srcf, dstf)


def _spmm_full(g, srcf, dstf, nchunk, c, n, n_pad):
    f = g.shape[1]
    fh = f // 2
    g2n = jnp.concatenate([g[:, :fh], g[:, fh:]], axis=0)
    out = _spmm(g2n, srcf, dstf, nchunk, c, n_pad).reshape(NC, n_pad, fh)[:, :n]
    return jnp.concatenate([out[0], out[1]], axis=1)


def _decode(z, a_r, b_r):
    n, f = z.shape
    nw, ncd, cd = a_r.shape

    @functools.partial(
        pl.kernel,
        out_type=jax.ShapeDtypeStruct((NW * ncd * cd,), jnp.float32),
        mesh=_mesh(),
        compiler_params=_SC_PARAMS,
        scratch_types=[
            pltpu.VMEM((ncd, cd), jnp.int32),
            pltpu.VMEM((ncd, cd), jnp.int32),
            pltpu.VMEM((cd, f), jnp.float32),
            pltpu.VMEM((cd, f), jnp.float32),
            pltpu.VMEM((cd,), jnp.float32),
            pltpu.SemaphoreType.DMA,
        ],
    )
    def decode(z_hbm, a_hbm, b_hbm, out_hbm, a_v, b_v, za_v, zb_v, lg_v, sem):
        cid = lax.axis_index("c")
        sid = lax.axis_index("s")
        wid = cid * NS + sid
        pltpu.sync_copy(a_hbm.at[wid], a_v)
        pltpu.sync_copy(b_hbm.at[wid], b_v)
        iota16 = lax.iota(jnp.int32, 16)

        def chunk(j, carry):
            pltpu.async_copy(z_hbm.at[a_v.at[j]], za_v, sem).wait()
            pltpu.async_copy(z_hbm.at[b_v.at[j]], zb_v, sem).wait()
            for grp in range(cd // 16):
                rows = iota16 + (16 * grp)

                def col8(t, acc):
                    base = jnp.full((16,), 8 * t, jnp.int32)
                    for k in range(8):
                        cols = base + k
                        acc = acc + plsc.load_gather(
                            za_v, [rows, cols]
                        ) * plsc.load_gather(zb_v, [rows, cols])
                    return acc

                lg_v[pl.ds(16 * grp, 16)] = lax.fori_loop(0, f // 8, col8, _Z16())
            pltpu.sync_copy(lg_v, out_hbm.at[pl.ds(wid * ncd * cd + j * cd, cd)])
            return carry

        lax.fori_loop(0, ncd, chunk, 0)

    return decode(z, a_r, b_r)


def _tc1(x, w1, d0, d1):
    n, k = x.shape
    h = w1.shape[1]
    r = 2000

    def body(x_ref, w_ref, d0_ref, d1_ref, g_ref, dinv_ref):
        deg = 1.0 + d0_ref[...] + d1_ref[...]
        dinv = lax.rsqrt(deg)
        hm = jnp.dot(x_ref[...], w_ref[...], preferred_element_type=jnp.float32)
        g_ref[...] = dinv * hm
        dinv_ref[...] = dinv

    return pl.pallas_call(
        body,
        grid=(n // r,),
        in_specs=[
            pl.BlockSpec((r, k), lambda i: (i, 0)),
            pl.BlockSpec((k, h), lambda i: (0, 0)),
            pl.BlockSpec((r, 1), lambda i: (i, 0)),
            pl.BlockSpec((r, 1), lambda i: (i, 0)),
        ],
        out_specs=[
            pl.BlockSpec((r, h), lambda i: (i, 0)),
            pl.BlockSpec((r, 1), lambda i: (i, 0)),
        ],
        out_shape=[
            jax.ShapeDtypeStruct((n, h), jnp.float32),
            jax.ShapeDtypeStruct((n, 1), jnp.float32),
        ],
    )(x, w1, d0, d1)


def _tc2(g1, s, dinv, b1, w2):
    n, h = g1.shape
    dout = w2.shape[1]
    r = 2000

    def body(g_ref, s_ref, di_ref, b_ref, w_ref, o_ref):
        di = di_ref[...]
        u = jnp.maximum(di * (s_ref[...] + g_ref[...]) + b_ref[...], 0.0)
        o_ref[...] = di * jnp.dot(u, w_ref[...], preferred_element_type=jnp.float32)

    return pl.pallas_call(
        body,
        grid=(n // r,),
        in_specs=[
            pl.BlockSpec((r, h), lambda i: (i, 0)),
            pl.BlockSpec((r, h), lambda i: (i, 0)),
            pl.BlockSpec((r, 1), lambda i: (i, 0)),
            pl.BlockSpec((1, h), lambda i: (0, 0)),
            pl.BlockSpec((h, dout), lambda i: (0, 0)),
        ],
        out_specs=pl.BlockSpec((r, dout), lambda i: (i, 0)),
        out_shape=jax.ShapeDtypeStruct((n, dout), jnp.float32),
    )(g1, s, dinv, b1, w2)


def _tc3(g2, s, dinv, b2):
    n, dout = g2.shape
    r = 2000

    def body(g_ref, s_ref, di_ref, b_ref, o_ref):
        o_ref[...] = di_ref[...] * (s_ref[...] + g_ref[...]) + b_ref[...]

    return pl.pallas_call(
        body,
        grid=(n // r,),
        in_specs=[
            pl.BlockSpec((r, dout), lambda i: (i, 0)),
            pl.BlockSpec((r, dout), lambda i: (i, 0)),
            pl.BlockSpec((r, 1), lambda i: (i, 0)),
            pl.BlockSpec((1, dout), lambda i: (0, 0)),
        ],
        out_specs=pl.BlockSpec((r, dout), lambda i: (i, 0)),
        out_shape=jax.ShapeDtypeStruct((n, dout), jnp.float32),
    )(g2, s, dinv, b2)


def kernel(x, edge_index, edge_label_index, W1, b1, W2, b2):
    n, _ = x.shape
    e = edge_index.shape[1]
    l = edge_label_index.shape[1]

    c = 80
    n_pad = 640 * NS

    ept_h = e // NW
    dst_r = edge_index[1].reshape(NW, ept_h // c, c)
    degp = _hist(dst_r, n_pad).reshape(NC, n_pad)
    d0 = degp[0, :n, None]
    d1 = degp[1, :n, None]

    ept = e // NS
    nchunk = ept // c
    src16 = edge_index[0].reshape(NS, nchunk, c)
    srcf = jnp.concatenate([src16, src16 + n], axis=0).reshape(NW * nchunk, c)
    dst16 = edge_index[1].reshape(NS, nchunk, c)
    dstf = jnp.concatenate([dst16, dst16], axis=0).reshape(NW * nchunk, c)

    g1, dinv = _tc1(x, W1, d0, d1)
    s1 = _spmm_full(g1, srcf, dstf, nchunk, c, n, n_pad)
    g2 = _tc2(g1, s1, dinv, b1.reshape(1, -1), W2)
    s2 = _spmm_full(g2, srcf, dstf, nchunk, c, n, n_pad)
    z = _tc3(g2, s2, dinv, b2.reshape(1, -1))

    cd = 64
    ppw = l // NW
    ncd = 10
    eli = edge_label_index.reshape(2, NW, ppw)
    eli = jnp.pad(eli, ((0, 0), (0, 0), (0, ncd * cd - ppw)))
    a_r = eli[0].reshape(NW, ncd, cd)
    b_r = eli[1].reshape(NW, ncd, cd)
    lp = _decode(z, a_r, b_r)
    return lp.reshape(NW, ncd * cd)[:, :ppw].reshape(l)


# speedup vs baseline: 25.0445x; 1.0764x over previous
import functools

import jax
import jax.numpy as jnp
from jax import lax
from jax.experimental import pallas as pl
from jax.experimental.pallas import tpu as pltpu
from jax.experimental.pallas import tpu_sc as plsc

NC = 2
NS = 16
NW = NC * NS


def _mesh():
    return plsc.VectorSubcoreMesh(
        core_axis_name="c", subcore_axis_name="s", num_cores=NC, num_subcores=NS
    )


_SC_PARAMS = pltpu.CompilerParams(
    use_tc_tiling_on_sc=False, needs_layout_passes=False
)


_Z16 = functools.partial(jnp.zeros, (16,), jnp.float32)


def _hist(dstf, nchunk, c, n_pad):
    per_tile = n_pad // NS
    kk = 5
    nblk = nchunk // kk

    @functools.partial(
        pl.kernel,
        out_type=jax.ShapeDtypeStruct((NC * n_pad,), jnp.float32),
        mesh=_mesh(),
        compiler_params=_SC_PARAMS,
        scratch_types=[
            pltpu.VMEM((2 * kk, c), jnp.int32),
            pltpu.VMEM((c,), jnp.float32),
            pltpu.VMEM((per_tile,), jnp.float32),
            pltpu.VMEM_SHARED((n_pad,), jnp.float32),
            pltpu.SemaphoreType.DMA,
        ],
    )
    def hist(dst_hbm, out_hbm, dst_ib, ones_v, zbuf_v, deg_sh, isem):
        cid = lax.axis_index("c")
        sid = lax.axis_index("s")
        wid = cid * NS + sid
        base = wid * nchunk
        for k in range(c // 16):
            ones_v[pl.ds(16 * k, 16)] = jnp.ones((16,), jnp.float32)
        for k in range(per_tile // 16):
            zbuf_v[pl.ds(16 * k, 16)] = _Z16()
        pltpu.sync_copy(zbuf_v, deg_sh.at[pl.ds(sid * per_tile, per_tile)])
        plsc.subcore_barrier()

        pltpu.sync_copy(dst_hbm.at[pl.ds(base, kk)], dst_ib.at[pl.ds(0, kk)])
        pltpu.async_copy(dst_hbm.at[pl.ds(base + kk, kk)], dst_ib.at[pl.ds(kk, kk)], isem)

        def body(t, carry):
            grp = lax.rem(t, 2) * kk
            for b in range(kk):
                pltpu.sync_copy(ones_v, deg_sh.at[dst_ib.at[grp + b]], add=True)

            @pl.when(t + 1 < nblk)
            def _():
                pltpu.make_async_copy(
                    dst_hbm.at[pl.ds(0, kk)], dst_ib.at[pl.ds(0, kk)], isem
                ).wait()

            @pl.when(t + 2 < nblk)
            def _():
                pltpu.async_copy(
                    dst_hbm.at[pl.ds(base + (t + 2) * kk, kk)],
                    dst_ib.at[pl.ds(grp, kk)], isem,
                )

            return carry

        lax.fori_loop(0, nblk, body, 0)
        plsc.subcore_barrier()
        pltpu.sync_copy(
            deg_sh.at[pl.ds(sid * per_tile, per_tile)],
            out_hbm.at[pl.ds(cid * n_pad + sid * per_tile, per_tile)],
        )

    return hist(dstf)


def _spmm(g2n, srcf, dstf, nchunk, c, n):
    _, fh = g2n.shape
    wr = 1000
    owners = n // wr
    zrows = 40
    kk = 5
    nblk = nchunk // kk

    @functools.partial(
        pl.kernel,
        out_type=[
            jax.ShapeDtypeStruct((n, fh), jnp.float32),
            jax.ShapeDtypeStruct((n, fh), jnp.float32),
        ],
        mesh=_mesh(),
        compiler_params=_SC_PARAMS,
        scratch_types=[
            pltpu.VMEM((3 * kk, c), jnp.int32),
            pltpu.VMEM((3 * kk, c), jnp.int32),
            pltpu.VMEM((2 * kk * c, fh), jnp.float32),
            pltpu.VMEM((zrows, fh), jnp.float32),
            pltpu.VMEM_SHARED((n, fh), jnp.float32),
            pltpu.SemaphoreType.DMA,
            pltpu.SemaphoreType.DMA,
            pltpu.SemaphoreType.DMA,
        ],
    )
    def spmm(g_hbm, src_hbm, dst_hbm, out0_hbm, out1_hbm, src_ib, dst_ib,
             rows_v, zbuf_v, acc_sh, gsem, ssem, isem):
        cid = lax.axis_index("c")
        sid = lax.axis_index("s")
        wid = cid * NS + sid
        base = wid * nchunk

        def zfill(i, carry):
            for k in range(fh // 16):
                zbuf_v[i, pl.ds(16 * k, 16)] = _Z16()
            return carry

        lax.fori_loop(0, zrows, zfill, 0)

        @pl.when(sid < owners)
        def _():
            def zcopy(k, carry):
                pltpu.sync_copy(zbuf_v, acc_sh.at[pl.ds(sid * wr + k * zrows, zrows)])
                return carry

            lax.fori_loop(0, wr // zrows, zcopy, 0)

        plsc.subcore_barrier()

        pltpu.sync_copy(src_hbm.at[pl.ds(base, kk)], src_ib.at[pl.ds(0, kk)])
        pltpu.sync_copy(dst_hbm.at[pl.ds(base, kk)], dst_ib.at[pl.ds(0, kk)])
        pltpu.async_copy(src_hbm.at[pl.ds(base + kk, kk)], src_ib.at[pl.ds(kk, kk)], isem)
        pltpu.async_copy(dst_hbm.at[pl.ds(base + kk, kk)], dst_ib.at[pl.ds(kk, kk)], isem)
        for b in range(kk):
            pltpu.async_copy(g_hbm.at[src_ib.at[b]], rows_v.at[pl.ds(b * c, c)], gsem)

        def block(t, carry):
            rg = lax.rem(t, 2) * kk
            g0 = lax.rem(t, 3) * kk
            g1 = lax.rem(t + 1, 3) * kk
            g2 = lax.rem(t + 2, 3) * kk
            for b in range(kk):
                pltpu.make_async_copy(
                    g_hbm.at[src_ib.at[0]], rows_v.at[pl.ds(0, c)], gsem
                ).wait()

            @pl.when(t >= 1)
            def _():
                for b in range(kk):
                    pltpu.make_async_copy(
                        g_hbm.at[src_ib.at[0]], rows_v.at[pl.ds(0, c)], ssem
                    ).wait()

            @pl.when(t + 2 < nblk)
            def _():
                pltpu.async_copy(
                    src_hbm.at[pl.ds(base + (t + 2) * kk, kk)],
                    src_ib.at[pl.ds(g2, kk)], isem,
                )
                pltpu.async_copy(
                    dst_hbm.at[pl.ds(base + (t + 2) * kk, kk)],
                    dst_ib.at[pl.ds(g2, kk)], isem,
                )

            for b in range(kk):
                pltpu.async_copy(
                    rows_v.at[pl.ds((rg + b) * c, c)],
                    acc_sh.at[dst_ib.at[g0 + b]], ssem, add=True,
                )

            @pl.when(t + 1 < nblk)
            def _():
                for b in range(2):
                    pltpu.make_async_copy(
                        src_hbm.at[pl.ds(0, kk)], src_ib.at[pl.ds(0, kk)], isem
                    ).wait()
                for b in range(kk):
                    pltpu.async_copy(
                        g_hbm.at[src_ib.at[g1 + b]],
                        rows_v.at[pl.ds((kk - rg + b) * c, c)], gsem,
                    )

            return carry

        lax.fori_loop(0, nblk, block, 0)
        for b in range(kk):
            pltpu.make_async_copy(
                g_hbm.at[src_ib.at[0]], rows_v.at[pl.ds(0, c)], ssem
            ).wait()
        plsc.subcore_barrier()

        @pl.when(sid < owners)
        def _():
            def wout(k, carry):
                r0 = sid * wr + k * 125
                sl = acc_sh.at[pl.ds(r0, 125)]

                @pl.when(cid == 0)
                def _():
                    pltpu.sync_copy(sl, out0_hbm.at[pl.ds(r0, 125)])

                @pl.when(cid == 1)
                def _():
                    pltpu.sync_copy(sl, out1_hbm.at[pl.ds(r0, 125)])

                return carry

            lax.fori_loop(0, wr // 125, wout, 0)

    return spmm(g2n, srcf, dstf)


def _decode(z, a_r, b_r):
    n, f = z.shape
    nw, ncd, cd = a_r.shape

    @functools.partial(
        pl.kernel,
        out_type=jax.ShapeDtypeStruct((NW * ncd * cd,), jnp.float32),
        mesh=_mesh(),
        compiler_params=_SC_PARAMS,
        scratch_types=[
            pltpu.VMEM((ncd, cd), jnp.int32),
            pltpu.VMEM((ncd, cd), jnp.int32),
            pltpu.VMEM((2 * cd, f), jnp.float32),
            pltpu.VMEM((2 * cd, f), jnp.float32),
            pltpu.VMEM((cd,), jnp.float32),
            pltpu.SemaphoreType.DMA,
        ],
    )
    def decode(z_hbm, a_hbm, b_hbm, out_hbm, a_v, b_v, za_v, zb_v, lg_v, sem):
        cid = lax.axis_index("c")
        sid = lax.axis_index("s")
        wid = cid * NS + sid
        pltpu.sync_copy(a_hbm.at[wid], a_v)
        pltpu.sync_copy(b_hbm.at[wid], b_v)
        iota16 = lax.iota(jnp.int32, 16)
        pltpu.async_copy(z_hbm.at[a_v.at[0]], za_v.at[pl.ds(0, cd)], sem)
        pltpu.async_copy(z_hbm.at[b_v.at[0]], zb_v.at[pl.ds(0, cd)], sem)

        def chunk(j, carry):
            buf = lax.rem(j, 2) * cd
            for _ in range(2):
                pltpu.make_async_copy(
                    z_hbm.at[a_v.at[0]], za_v.at[pl.ds(0, cd)], sem
                ).wait()

            @pl.when(j + 1 < ncd)
            def _():
                nbuf = cd - buf
                pltpu.async_copy(z_hbm.at[a_v.at[j + 1]], za_v.at[pl.ds(nbuf, cd)], sem)
                pltpu.async_copy(z_hbm.at[b_v.at[j + 1]], zb_v.at[pl.ds(nbuf, cd)], sem)

            for grp in range(cd // 16):
                rows = iota16 + (16 * grp) + buf

                def col8(t, acc):
                    base = jnp.full((16,), 8 * t, jnp.int32)
                    for k in range(8):
                        cols = base + k
                        acc = acc + plsc.load_gather(
                            za_v, [rows, cols]
                        ) * plsc.load_gather(zb_v, [rows, cols])
                    return acc

                lg_v[pl.ds(16 * grp, 16)] = lax.fori_loop(0, f // 8, col8, _Z16())
            pltpu.sync_copy(lg_v, out_hbm.at[pl.ds(wid * ncd * cd + j * cd, cd)])
            return carry

        lax.fori_loop(0, ncd, chunk, 0)

    return decode(z, a_r, b_r)


def _tc1(x, w1, d0, d1):
    n, k = x.shape
    hh = w1.shape[2]
    r = 2000
    gi = n // r

    def body(x_ref, w_ref, d0_ref, d1_ref, g_ref, dinv_ref):
        deg = 1.0 + d0_ref[...] + d1_ref[...]
        dinv = lax.rsqrt(deg)
        hm = jnp.dot(x_ref[...], w_ref[0], preferred_element_type=jnp.float32)
        g_ref[...] = dinv * hm
        dinv_ref[...] = dinv

    return pl.pallas_call(
        body,
        grid=(gi, 2),
        in_specs=[
            pl.BlockSpec((r, k), lambda i, j: (i, 0)),
            pl.BlockSpec((1, k, hh), lambda i, j: (j, 0, 0)),
            pl.BlockSpec((r, 1), lambda i, j: (i, 0)),
            pl.BlockSpec((r, 1), lambda i, j: (i, 0)),
        ],
        out_specs=[
            pl.BlockSpec((r, hh), lambda i, j: (j * gi + i, 0)),
            pl.BlockSpec((r, 1), lambda i, j: (i, 0)),
        ],
        out_shape=[
            jax.ShapeDtypeStruct((2 * n, hh), jnp.float32),
            jax.ShapeDtypeStruct((n, 1), jnp.float32),
        ],
    )(x, w1, d0, d1)


def _tc2(g1s, s0, s1, dinv, b1, w2):
    n2, hh = g1s.shape
    n = n2 // 2
    h = 2 * hh
    dh = w2.shape[2]
    r = 2000
    gi = n // r

    def body(gl_ref, gr_ref, s0_ref, s1_ref, di_ref, b_ref, w_ref, o_ref):
        di = di_ref[...]
        ul = jnp.maximum(di * (s0_ref[...] + gl_ref[...]) + b_ref[:, :hh], 0.0)
        ur = jnp.maximum(di * (s1_ref[...] + gr_ref[...]) + b_ref[:, hh:], 0.0)
        u = jnp.concatenate([ul, ur], axis=1)
        o_ref[...] = di * jnp.dot(u, w_ref[0], preferred_element_type=jnp.float32)

    return pl.pallas_call(
        body,
        grid=(gi, 2),
        in_specs=[
            pl.BlockSpec((r, hh), lambda i, j: (i, 0)),
            pl.BlockSpec((r, hh), lambda i, j: (gi + i, 0)),
            pl.BlockSpec((r, hh), lambda i, j: (i, 0)),
            pl.BlockSpec((r, hh), lambda i, j: (i, 0)),
            pl.BlockSpec((r, 1), lambda i, j: (i, 0)),
            pl.BlockSpec((1, h), lambda i, j: (0, 0)),
            pl.BlockSpec((1, h, dh), lambda i, j: (j, 0, 0)),
        ],
        out_specs=pl.BlockSpec((r, dh), lambda i, j: (j * gi + i, 0)),
        out_shape=jax.ShapeDtypeStruct((2 * n, dh), jnp.float32),
    )(g1s, g1s, s0, s1, dinv, b1, w2)


def _tc3(g2s, s0, s1, dinv, b2):
    n2, dh = g2s.shape
    n = n2 // 2
    dout = 2 * dh
    r = 2000
    gi = n // r

    def body(gl_ref, gr_ref, s0_ref, s1_ref, di_ref, b_ref, o_ref):
        di = di_ref[...]
        zl = di * (s0_ref[...] + gl_ref[...]) + b_ref[:, :dh]
        zr = di * (s1_ref[...] + gr_ref[...]) + b_ref[:, dh:]
        o_ref[...] = jnp.concatenate([zl, zr], axis=1)

    return pl.pallas_call(
        body,
        grid=(gi,),
        in_specs=[
            pl.BlockSpec((r, dh), lambda i: (i, 0)),
            pl.BlockSpec((r, dh), lambda i: (gi + i, 0)),
            pl.BlockSpec((r, dh), lambda i: (i, 0)),
            pl.BlockSpec((r, dh), lambda i: (i, 0)),
            pl.BlockSpec((r, 1), lambda i: (i, 0)),
            pl.BlockSpec((1, dout), lambda i: (0, 0)),
        ],
        out_specs=pl.BlockSpec((r, dout), lambda i: (i, 0)),
        out_shape=jax.ShapeDtypeStruct((n, dout), jnp.float32),
    )(g2s, g2s, s0, s1, dinv, b2)


def kernel(x, edge_index, edge_label_index, W1, b1, W2, b2):
    n, _ = x.shape
    e = edge_index.shape[1]
    l = edge_label_index.shape[1]

    c = 80
    n_pad = 640 * NS

    ept_h = e // NW
    dsth = edge_index[1].reshape(NW * (ept_h // c), c)
    degp = _hist(dsth, ept_h // c, c, n_pad).reshape(NC, n_pad)
    d0 = degp[0, :n, None]
    d1 = degp[1, :n, None]

    ept = e // NS
    nchunk = ept // c
    src16 = edge_index[0].reshape(NS, nchunk, c)
    srcf = jnp.concatenate([src16, src16 + n], axis=0).reshape(NW * nchunk, c)
    dst16 = edge_index[1].reshape(NS, nchunk, c)
    dstf = jnp.concatenate([dst16, dst16], axis=0).reshape(NW * nchunk, c)

    hh = W1.shape[1] // 2
    dh = W2.shape[1] // 2
    w1stk = jnp.stack([W1[:, :hh], W1[:, hh:]])
    w2stk = jnp.stack([W2[:, :dh], W2[:, dh:]])

    g1s, dinv = _tc1(x, w1stk, d0, d1)
    s1l, s1r = _spmm(g1s, srcf, dstf, nchunk, c, n)
    g2s = _tc2(g1s, s1l, s1r, dinv, b1.reshape(1, -1), w2stk)
    s2l, s2r = _spmm(g2s, srcf, dstf, nchunk, c, n)
    z = _tc3(g2s, s2l, s2r, dinv, b2.reshape(1, -1))

    cd = 64
    ppw = l // NW
    ncd = 10
    eli = edge_label_index.reshape(2, NW, ppw)
    eli = jnp.pad(eli, ((0, 0), (0, 0), (0, ncd * cd - ppw)))
    a_r = eli[0].reshape(NW, ncd, cd)
    b_r = eli[1].reshape(NW, ncd, cd)
    lp = _decode(z, a_r, b_r)
    return lp.reshape(NW, ncd * cd)[:, :ppw].reshape(l)
